# baseline (speedup 1.0000x reference)
import jax
import jax.numpy as jnp
from jax.experimental import pallas as pl
from jax.experimental.pallas import tpu as pltpu

_COLS = 12800


def _stats_tail_kernel(w_ref, k_ref, wt_ref, xt_ref, stats_ref, tail_ref):
    wv = w_ref[...]
    m = jnp.max(wv)
    d = jnp.sum(jnp.exp(wv - m))
    coeff = jnp.round(k_ref[0, 0]) / (jnp.float32(wv.size) * d)
    stats_ref[...] = jnp.stack([jnp.full((16,), m), jnp.full((16,), coeff)])
    e_t = jnp.exp(wt_ref[...] - m) * coeff
    tail_ref[...] = jax.lax.dot_general(
        e_t, xt_ref[...], (((1,), (0,)), ((), ())),
        preferred_element_type=jnp.float32)


def _stats_kernel(w_ref, k_ref, out_ref):
    wv = w_ref[...]
    m = jnp.max(wv)
    d = jnp.sum(jnp.exp(wv - m))
    coeff = jnp.round(k_ref[0, 0]) / (jnp.float32(wv.size) * d)
    out_ref[...] = jnp.stack([jnp.full((16,), m), jnp.full((16,), coeff)])


def _wsum_tc_grid_kernel(stats_ref, w_ref, x_ref, out_ref):
    i = pl.program_id(0)
    m = stats_ref[0, 0]
    coeff = stats_ref[1, 0]
    e = jnp.exp(w_ref[0] - m) * coeff
    part = jax.lax.dot_general(
        e, x_ref[...], (((1,), (0,)), ((), ())),
        preferred_element_type=jnp.float32)

    @pl.when(i == 0)
    def _init():
        out_ref[...] = jnp.zeros_like(out_ref)

    out_ref[...] += part


def _make_xt_kernel(n, t, b, nb):
    def body(stats_ref, tailp_ref, w_ref, xt_ref, out_ref,
             xa, xb_, wa, wb, sb, tb, acc, sxa, sxb, swa, swb):
        pltpu.sync_copy(stats_ref, sb)
        pltpu.sync_copy(tailp_ref, tb)
        m = sb[0, 0]
        coeff = sb[1, 0]

        xbufs = (xa, xb_)
        wbufs = (wa, wb)
        sxs = (sxa, sxb)
        sws = (swa, swb)

        def start(j, p):
            pltpu.async_copy(xt_ref.at[:, pl.ds(j * b, b)], xbufs[p], sxs[p])
            pltpu.async_copy(w_ref.at[pl.ds(j * b, b)], wbufs[p], sws[p])

        def wait(p):
            pltpu.make_async_copy(
                xt_ref.at[:, pl.ds(0, b)], xbufs[p], sxs[p]).wait()
            pltpu.make_async_copy(
                w_ref.at[pl.ds(0, b)], wbufs[p], sws[p]).wait()

        def process(p):
            e = (jnp.exp(wbufs[p][...] - m) * coeff).reshape(1, b)
            acc[...] += xbufs[p][...] * e

        start(0, 0)
        if nb > 1:
            start(1, 1)
        acc[...] = jnp.zeros_like(acc)

        def pair(i, carry):
            j0 = 2 * i
            wait(0)
            process(0)

            @pl.when(j0 + 2 < nb)
            def _():
                start(j0 + 2, 0)

            wait(1)
            process(1)

            @pl.when(j0 + 3 < nb)
            def _():
                start(j0 + 3, 1)

            return carry

        jax.lax.fori_loop(0, nb // 2, pair, 0)
        if nb % 2 == 1:
            wait((nb - 1) % 2)
            process((nb - 1) % 2)
        out_ref[...] = jnp.sum(acc[...], axis=1).reshape(1, t) + tb[...]

    return pl.pallas_call(
        body,
        out_shape=jax.ShapeDtypeStruct((1, t), jnp.float32),
        in_specs=[
            pl.BlockSpec(memory_space=pltpu.HBM),
            pl.BlockSpec(memory_space=pltpu.HBM),
            pl.BlockSpec(memory_space=pltpu.HBM),
            pl.BlockSpec(memory_space=pltpu.HBM),
        ],
        out_specs=pl.BlockSpec((1, t), lambda: (0, 0)),
        scratch_shapes=[
            pltpu.VMEM((t, b), jnp.float32),
            pltpu.VMEM((t, b), jnp.float32),
            pltpu.VMEM((b,), jnp.float32),
            pltpu.VMEM((b,), jnp.float32),
            pltpu.VMEM((2, 16), jnp.float32),
            pltpu.VMEM((1, t), jnp.float32),
            pltpu.VMEM((t, b), jnp.float32),
            pltpu.SemaphoreType.DMA,
            pltpu.SemaphoreType.DMA,
            pltpu.SemaphoreType.DMA,
            pltpu.SemaphoreType.DMA,
        ],
    )


def _pick_block(n):
    for b in (8000, 10000, 5000, 4096, 4000, 2048, 2000, 1000):
        if n % b == 0:
            return b
    return n


def kernel(x, w, k_param):
    n, t = x.shape
    rows = 1000 if n % 1000 == 0 else 1
    w2d = w.reshape(n // rows, rows)
    k2d = k_param.reshape(1, 1)

    bc = _COLS
    nb = n // bc
    tail = n - nb * bc
    use_xt = (t % 8 == 0 and nb >= 2 and tail % 8 == 0 and tail > 0
              and bc % 1024 == 0)

    if use_xt:
        wt = w[n - tail:].reshape(1, tail)
        xtail = x[n - tail:]
        stats, tailp = pl.pallas_call(
            _stats_tail_kernel,
            out_shape=(
                jax.ShapeDtypeStruct((2, 16), jnp.float32),
                jax.ShapeDtypeStruct((1, t), jnp.float32),
            ),
            in_specs=[
                pl.BlockSpec((n // rows, rows), lambda: (0, 0)),
                pl.BlockSpec((1, 1), lambda: (0, 0)),
                pl.BlockSpec((1, tail), lambda: (0, 0)),
                pl.BlockSpec((tail, t), lambda: (0, 0)),
            ],
            out_specs=(
                pl.BlockSpec((2, 16), lambda: (0, 0)),
                pl.BlockSpec((1, t), lambda: (0, 0)),
            ),
        )(w2d, k2d, wt, xtail)
        out = _make_xt_kernel(n, t, bc, nb)(stats, tailp, w, x.T)
    else:
        stats = pl.pallas_call(
            _stats_kernel,
            out_shape=jax.ShapeDtypeStruct((2, 16), jnp.float32),
            in_specs=[
                pl.BlockSpec((n // rows, rows), lambda: (0, 0)),
                pl.BlockSpec((1, 1), lambda: (0, 0)),
            ],
            out_specs=pl.BlockSpec((2, 16), lambda: (0, 0)),
        )(w2d, k2d)
        b = _pick_block(n)
        out = pl.pallas_call(
            _wsum_tc_grid_kernel,
            grid=(n // b,),
            out_shape=jax.ShapeDtypeStruct((1, t), jnp.float32),
            in_specs=[
                pl.BlockSpec((2, 16), lambda i: (0, 0)),
                pl.BlockSpec((1, 1, b), lambda i: (i, 0, 0)),
                pl.BlockSpec((b, t), lambda i: (i, 0)),
            ],
            out_specs=pl.BlockSpec((1, t), lambda i: (0, 0)),
        )(stats, w.reshape(n // b, 1, b), x)

    return out.reshape(t)


# speedup vs baseline: 4.5905x; 4.5905x over previous
import jax
import jax.numpy as jnp
from jax.experimental import pallas as pl
from jax.experimental.pallas import tpu as pltpu

_COLS = 51200


def _stats_tail_kernel(w_ref, k_ref, wt_ref, xt_ref, stats_ref, tail_ref):
    wv = w_ref[...]
    m = jnp.max(wv)
    d = jnp.sum(jnp.exp(wv - m))
    coeff = jnp.round(k_ref[0, 0]) / (jnp.float32(wv.size) * d)
    stats_ref[...] = jnp.stack([jnp.full((16,), m), jnp.full((16,), coeff)])
    e_t = jnp.exp(wt_ref[...] - m) * coeff
    tail_ref[...] = jax.lax.dot_general(
        e_t, xt_ref[...], (((1,), (0,)), ((), ())),
        preferred_element_type=jnp.float32)


def _stats_kernel(w_ref, k_ref, out_ref):
    wv = w_ref[...]
    m = jnp.max(wv)
    d = jnp.sum(jnp.exp(wv - m))
    coeff = jnp.round(k_ref[0, 0]) / (jnp.float32(wv.size) * d)
    out_ref[...] = jnp.stack([jnp.full((16,), m), jnp.full((16,), coeff)])


def _wsum_tc_grid_kernel(stats_ref, w_ref, x_ref, out_ref):
    i = pl.program_id(0)
    m = stats_ref[0, 0]
    coeff = stats_ref[1, 0]
    e = jnp.exp(w_ref[0] - m) * coeff
    part = jax.lax.dot_general(
        e, x_ref[...], (((1,), (0,)), ((), ())),
        preferred_element_type=jnp.float32)

    @pl.when(i == 0)
    def _init():
        out_ref[...] = jnp.zeros_like(out_ref)

    out_ref[...] += part


def _make_xt_kernel(n, t, b, nb):
    def body(stats_ref, tailp_ref, w_ref, xt_ref, out_ref,
             xa, xb_, wa, wb, sb, tb, acc, sxa, sxb, swa, swb):
        pltpu.sync_copy(stats_ref, sb)
        pltpu.sync_copy(tailp_ref, tb)
        m = sb[0, 0]
        coeff = sb[1, 0]

        xbufs = (xa, xb_)
        wbufs = (wa, wb)
        sxs = (sxa, sxb)
        sws = (swa, swb)

        def start(j, p):
            pltpu.async_copy(xt_ref.at[:, pl.ds(j * b, b)], xbufs[p], sxs[p])
            pltpu.async_copy(w_ref.at[pl.ds(j * b, b)], wbufs[p], sws[p])

        def wait(p):
            pltpu.make_async_copy(
                xt_ref.at[:, pl.ds(0, b)], xbufs[p], sxs[p]).wait()
            pltpu.make_async_copy(
                w_ref.at[pl.ds(0, b)], wbufs[p], sws[p]).wait()

        def process(p):
            e = (jnp.exp(wbufs[p][...] - m) * coeff).reshape(1, b)
            acc[...] += xbufs[p][...] * e

        start(0, 0)
        if nb > 1:
            start(1, 1)
        acc[...] = jnp.zeros_like(acc)

        def pair(i, carry):
            j0 = 2 * i
            wait(0)
            process(0)

            @pl.when(j0 + 2 < nb)
            def _():
                start(j0 + 2, 0)

            wait(1)
            process(1)

            @pl.when(j0 + 3 < nb)
            def _():
                start(j0 + 3, 1)

            return carry

        jax.lax.fori_loop(0, nb // 2, pair, 0)
        if nb % 2 == 1:
            wait((nb - 1) % 2)
            process((nb - 1) % 2)
        out_ref[...] = jnp.sum(acc[...], axis=1).reshape(1, t) + tb[...]

    return pl.pallas_call(
        body,
        out_shape=jax.ShapeDtypeStruct((1, t), jnp.float32),
        in_specs=[
            pl.BlockSpec(memory_space=pltpu.HBM),
            pl.BlockSpec(memory_space=pltpu.HBM),
            pl.BlockSpec(memory_space=pltpu.HBM),
            pl.BlockSpec(memory_space=pltpu.HBM),
        ],
        out_specs=pl.BlockSpec((1, t), lambda: (0, 0)),
        scratch_shapes=[
            pltpu.VMEM((t, b), jnp.float32),
            pltpu.VMEM((t, b), jnp.float32),
            pltpu.VMEM((b,), jnp.float32),
            pltpu.VMEM((b,), jnp.float32),
            pltpu.VMEM((2, 16), jnp.float32),
            pltpu.VMEM((1, t), jnp.float32),
            pltpu.VMEM((t, b), jnp.float32),
            pltpu.SemaphoreType.DMA,
            pltpu.SemaphoreType.DMA,
            pltpu.SemaphoreType.DMA,
            pltpu.SemaphoreType.DMA,
        ],
    )


def _pick_block(n):
    for b in (8000, 10000, 5000, 4096, 4000, 2048, 2000, 1000):
        if n % b == 0:
            return b
    return n


def kernel(x, w, k_param):
    n, t = x.shape
    rows = 1000 if n % 1000 == 0 else 1
    w2d = w.reshape(n // rows, rows)
    k2d = k_param.reshape(1, 1)

    bc = _COLS
    nb = n // bc
    tail = n - nb * bc
    use_xt = (t % 8 == 0 and nb >= 2 and tail % 8 == 0 and tail > 0
              and bc % 1024 == 0)

    if use_xt:
        wt = w[n - tail:].reshape(1, tail)
        xtail = x[n - tail:]
        stats, tailp = pl.pallas_call(
            _stats_tail_kernel,
            out_shape=(
                jax.ShapeDtypeStruct((2, 16), jnp.float32),
                jax.ShapeDtypeStruct((1, t), jnp.float32),
            ),
            in_specs=[
                pl.BlockSpec((n // rows, rows), lambda: (0, 0)),
                pl.BlockSpec((1, 1), lambda: (0, 0)),
                pl.BlockSpec((1, tail), lambda: (0, 0)),
                pl.BlockSpec((tail, t), lambda: (0, 0)),
            ],
            out_specs=(
                pl.BlockSpec((2, 16), lambda: (0, 0)),
                pl.BlockSpec((1, t), lambda: (0, 0)),
            ),
        )(w2d, k2d, wt, xtail)
        out = _make_xt_kernel(n, t, bc, nb)(stats, tailp, w, x.T)
    else:
        stats = pl.pallas_call(
            _stats_kernel,
            out_shape=jax.ShapeDtypeStruct((2, 16), jnp.float32),
            in_specs=[
                pl.BlockSpec((n // rows, rows), lambda: (0, 0)),
                pl.BlockSpec((1, 1), lambda: (0, 0)),
            ],
            out_specs=pl.BlockSpec((2, 16), lambda: (0, 0)),
        )(w2d, k2d)
        b = _pick_block(n)
        out = pl.pallas_call(
            _wsum_tc_grid_kernel,
            grid=(n // b,),
            out_shape=jax.ShapeDtypeStruct((1, t), jnp.float32),
            in_specs=[
                pl.BlockSpec((2, 16), lambda i: (0, 0)),
                pl.BlockSpec((1, 1, b), lambda i: (i, 0, 0)),
                pl.BlockSpec((b, t), lambda i: (i, 0)),
            ],
            out_specs=pl.BlockSpec((1, t), lambda i: (0, 0)),
        )(stats, w.reshape(n // b, 1, b), x)

    return out.reshape(t)


# speedup vs baseline: 4.9064x; 1.0688x over previous
import jax
import jax.numpy as jnp
from jax.experimental import pallas as pl
from jax.experimental.pallas import tpu as pltpu

_COLS = 20480


def _stats_tail_kernel(w_ref, k_ref, wt_ref, xt_ref, stats_ref, tail_ref):
    wv = w_ref[...]
    m = jnp.max(wv)
    d = jnp.sum(jnp.exp(wv - m))
    coeff = jnp.round(k_ref[0, 0]) / (jnp.float32(wv.size) * d)
    stats_ref[...] = jnp.stack([jnp.full((16,), m), jnp.full((16,), coeff)])
    e_t = jnp.exp(wt_ref[...] - m) * coeff
    tail_ref[...] = jax.lax.dot_general(
        e_t, xt_ref[...], (((1,), (0,)), ((), ())),
        preferred_element_type=jnp.float32)


def _stats_kernel(w_ref, k_ref, out_ref):
    wv = w_ref[...]
    m = jnp.max(wv)
    d = jnp.sum(jnp.exp(wv - m))
    coeff = jnp.round(k_ref[0, 0]) / (jnp.float32(wv.size) * d)
    out_ref[...] = jnp.stack([jnp.full((16,), m), jnp.full((16,), coeff)])


def _wsum_tc_grid_kernel(stats_ref, w_ref, x_ref, out_ref):
    i = pl.program_id(0)
    m = stats_ref[0, 0]
    coeff = stats_ref[1, 0]
    e = jnp.exp(w_ref[0] - m) * coeff
    part = jax.lax.dot_general(
        e, x_ref[...], (((1,), (0,)), ((), ())),
        preferred_element_type=jnp.float32)

    @pl.when(i == 0)
    def _init():
        out_ref[...] = jnp.zeros_like(out_ref)

    out_ref[...] += part


def _make_xt_kernel(n, t, b, nb):
    def body(stats_ref, tailp_ref, w_ref, xt_ref, out_ref,
             xa, xb_, wa, wb, sb, tb, acc, sxa, sxb, swa, swb):
        pltpu.sync_copy(stats_ref, sb)
        pltpu.sync_copy(tailp_ref, tb)
        m = sb[0, 0]
        coeff = sb[1, 0]

        xbufs = (xa, xb_)
        wbufs = (wa, wb)
        sxs = (sxa, sxb)
        sws = (swa, swb)

        def start(j, p):
            pltpu.async_copy(xt_ref.at[:, pl.ds(j * b, b)], xbufs[p], sxs[p])
            pltpu.async_copy(w_ref.at[pl.ds(j * b, b)], wbufs[p], sws[p])

        def wait(p):
            pltpu.make_async_copy(
                xt_ref.at[:, pl.ds(0, b)], xbufs[p], sxs[p]).wait()
            pltpu.make_async_copy(
                w_ref.at[pl.ds(0, b)], wbufs[p], sws[p]).wait()

        def process(p):
            e = (jnp.exp(wbufs[p][...] - m) * coeff).reshape(1, b)
            acc[...] += xbufs[p][...] * e

        start(0, 0)
        if nb > 1:
            start(1, 1)
        acc[...] = jnp.zeros_like(acc)

        def pair(i, carry):
            j0 = 2 * i
            wait(0)
            process(0)

            @pl.when(j0 + 2 < nb)
            def _():
                start(j0 + 2, 0)

            wait(1)
            process(1)

            @pl.when(j0 + 3 < nb)
            def _():
                start(j0 + 3, 1)

            return carry

        jax.lax.fori_loop(0, nb // 2, pair, 0)
        if nb % 2 == 1:
            wait((nb - 1) % 2)
            process((nb - 1) % 2)
        out_ref[...] = jnp.sum(acc[...], axis=1).reshape(1, t) + tb[...]

    return pl.pallas_call(
        body,
        out_shape=jax.ShapeDtypeStruct((1, t), jnp.float32),
        in_specs=[
            pl.BlockSpec(memory_space=pltpu.HBM),
            pl.BlockSpec(memory_space=pltpu.HBM),
            pl.BlockSpec(memory_space=pltpu.HBM),
            pl.BlockSpec(memory_space=pltpu.HBM),
        ],
        out_specs=pl.BlockSpec((1, t), lambda: (0, 0)),
        scratch_shapes=[
            pltpu.VMEM((t, b), jnp.float32),
            pltpu.VMEM((t, b), jnp.float32),
            pltpu.VMEM((b,), jnp.float32),
            pltpu.VMEM((b,), jnp.float32),
            pltpu.VMEM((2, 16), jnp.float32),
            pltpu.VMEM((1, t), jnp.float32),
            pltpu.VMEM((t, b), jnp.float32),
            pltpu.SemaphoreType.DMA,
            pltpu.SemaphoreType.DMA,
            pltpu.SemaphoreType.DMA,
            pltpu.SemaphoreType.DMA,
        ],
    )


def _pick_block(n):
    for b in (8000, 10000, 5000, 4096, 4000, 2048, 2000, 1000):
        if n % b == 0:
            return b
    return n


def kernel(x, w, k_param):
    n, t = x.shape
    rows = 1000 if n % 1000 == 0 else 1
    w2d = w.reshape(n // rows, rows)
    k2d = k_param.reshape(1, 1)

    bc = _COLS
    nb = n // bc
    tail = n - nb * bc
    use_xt = (t % 8 == 0 and nb >= 2 and tail % 8 == 0 and tail > 0
              and bc % 1024 == 0)

    if use_xt:
        wt = w[n - tail:].reshape(1, tail)
        xtail = x[n - tail:]
        stats, tailp = pl.pallas_call(
            _stats_tail_kernel,
            out_shape=(
                jax.ShapeDtypeStruct((2, 16), jnp.float32),
                jax.ShapeDtypeStruct((1, t), jnp.float32),
            ),
            in_specs=[
                pl.BlockSpec((n // rows, rows), lambda: (0, 0)),
                pl.BlockSpec((1, 1), lambda: (0, 0)),
                pl.BlockSpec((1, tail), lambda: (0, 0)),
                pl.BlockSpec((tail, t), lambda: (0, 0)),
            ],
            out_specs=(
                pl.BlockSpec((2, 16), lambda: (0, 0)),
                pl.BlockSpec((1, t), lambda: (0, 0)),
            ),
        )(w2d, k2d, wt, xtail)
        out = _make_xt_kernel(n, t, bc, nb)(stats, tailp, w, x.T)
    else:
        stats = pl.pallas_call(
            _stats_kernel,
            out_shape=jax.ShapeDtypeStruct((2, 16), jnp.float32),
            in_specs=[
                pl.BlockSpec((n // rows, rows), lambda: (0, 0)),
                pl.BlockSpec((1, 1), lambda: (0, 0)),
            ],
            out_specs=pl.BlockSpec((2, 16), lambda: (0, 0)),
        )(w2d, k2d)
        b = _pick_block(n)
        out = pl.pallas_call(
            _wsum_tc_grid_kernel,
            grid=(n // b,),
            out_shape=jax.ShapeDtypeStruct((1, t), jnp.float32),
            in_specs=[
                pl.BlockSpec((2, 16), lambda i: (0, 0)),
                pl.BlockSpec((1, 1, b), lambda i: (i, 0, 0)),
                pl.BlockSpec((b, t), lambda i: (i, 0)),
            ],
            out_specs=pl.BlockSpec((1, t), lambda i: (0, 0)),
        )(stats, w.reshape(n // b, 1, b), x)

    return out.reshape(t)


# speedup vs baseline: 5.3989x; 1.1004x over previous
import jax
import jax.numpy as jnp
from jax.experimental import pallas as pl
from jax.experimental.pallas import tpu as pltpu

_COLS = 25600


def _stats_tail_kernel(w_ref, k_ref, wt_ref, xt_ref, stats_ref, tail_ref):
    wv = w_ref[...]
    m = jnp.max(wv)
    d = jnp.sum(jnp.exp(wv - m))
    coeff = jnp.round(k_ref[0, 0]) / (jnp.float32(wv.size) * d)
    stats_ref[...] = jnp.stack([jnp.full((16,), m), jnp.full((16,), coeff)])
    e_t = jnp.exp(wt_ref[...] - m) * coeff
    tail_ref[...] = jax.lax.dot_general(
        e_t, xt_ref[...], (((1,), (0,)), ((), ())),
        preferred_element_type=jnp.float32)


def _stats_kernel(w_ref, k_ref, out_ref):
    wv = w_ref[...]
    m = jnp.max(wv)
    d = jnp.sum(jnp.exp(wv - m))
    coeff = jnp.round(k_ref[0, 0]) / (jnp.float32(wv.size) * d)
    out_ref[...] = jnp.stack([jnp.full((16,), m), jnp.full((16,), coeff)])


def _wsum_tc_grid_kernel(stats_ref, w_ref, x_ref, out_ref):
    i = pl.program_id(0)
    m = stats_ref[0, 0]
    coeff = stats_ref[1, 0]
    e = jnp.exp(w_ref[0] - m) * coeff
    part = jax.lax.dot_general(
        e, x_ref[...], (((1,), (0,)), ((), ())),
        preferred_element_type=jnp.float32)

    @pl.when(i == 0)
    def _init():
        out_ref[...] = jnp.zeros_like(out_ref)

    out_ref[...] += part


def _make_xt_kernel(n, t, b, nb):
    def body(stats_ref, tailp_ref, w_ref, xt_ref, out_ref,
             xa, xb_, wa, wb, sb, tb, acc, sxa, sxb, swa, swb):
        pltpu.sync_copy(stats_ref, sb)
        pltpu.sync_copy(tailp_ref, tb)
        m = sb[0, 0]
        coeff = sb[1, 0]

        xbufs = (xa, xb_)
        wbufs = (wa, wb)
        sxs = (sxa, sxb)
        sws = (swa, swb)

        def start(j, p):
            pltpu.async_copy(xt_ref.at[:, pl.ds(j * b, b)], xbufs[p], sxs[p])
            pltpu.async_copy(w_ref.at[pl.ds(j * b, b)], wbufs[p], sws[p])

        def wait(p):
            pltpu.make_async_copy(
                xt_ref.at[:, pl.ds(0, b)], xbufs[p], sxs[p]).wait()
            pltpu.make_async_copy(
                w_ref.at[pl.ds(0, b)], wbufs[p], sws[p]).wait()

        def process(p):
            e = (jnp.exp(wbufs[p][...] - m) * coeff).reshape(1, b)
            acc[...] += xbufs[p][...] * e

        start(0, 0)
        if nb > 1:
            start(1, 1)
        acc[...] = jnp.zeros_like(acc)

        def pair(i, carry):
            j0 = 2 * i
            wait(0)
            process(0)

            @pl.when(j0 + 2 < nb)
            def _():
                start(j0 + 2, 0)

            wait(1)
            process(1)

            @pl.when(j0 + 3 < nb)
            def _():
                start(j0 + 3, 1)

            return carry

        jax.lax.fori_loop(0, nb // 2, pair, 0)
        if nb % 2 == 1:
            wait((nb - 1) % 2)
            process((nb - 1) % 2)
        out_ref[...] = jnp.sum(acc[...], axis=1).reshape(1, t) + tb[...]

    return pl.pallas_call(
        body,
        out_shape=jax.ShapeDtypeStruct((1, t), jnp.float32),
        in_specs=[
            pl.BlockSpec(memory_space=pltpu.HBM),
            pl.BlockSpec(memory_space=pltpu.HBM),
            pl.BlockSpec(memory_space=pltpu.HBM),
            pl.BlockSpec(memory_space=pltpu.HBM),
        ],
        out_specs=pl.BlockSpec((1, t), lambda: (0, 0)),
        scratch_shapes=[
            pltpu.VMEM((t, b), jnp.float32),
            pltpu.VMEM((t, b), jnp.float32),
            pltpu.VMEM((b,), jnp.float32),
            pltpu.VMEM((b,), jnp.float32),
            pltpu.VMEM((2, 16), jnp.float32),
            pltpu.VMEM((1, t), jnp.float32),
            pltpu.VMEM((t, b), jnp.float32),
            pltpu.SemaphoreType.DMA,
            pltpu.SemaphoreType.DMA,
            pltpu.SemaphoreType.DMA,
            pltpu.SemaphoreType.DMA,
        ],
    )


def _pick_block(n):
    for b in (8000, 10000, 5000, 4096, 4000, 2048, 2000, 1000):
        if n % b == 0:
            return b
    return n


def kernel(x, w, k_param):
    n, t = x.shape
    rows = 1000 if n % 1000 == 0 else 1
    w2d = w.reshape(n // rows, rows)
    k2d = k_param.reshape(1, 1)

    bc = _COLS
    nb = n // bc
    tail = n - nb * bc
    use_xt = (t % 8 == 0 and nb >= 2 and tail % 8 == 0 and tail > 0
              and bc % 1024 == 0)

    if use_xt:
        wt = w[n - tail:].reshape(1, tail)
        xtail = x[n - tail:]
        stats, tailp = pl.pallas_call(
            _stats_tail_kernel,
            out_shape=(
                jax.ShapeDtypeStruct((2, 16), jnp.float32),
                jax.ShapeDtypeStruct((1, t), jnp.float32),
            ),
            in_specs=[
                pl.BlockSpec((n // rows, rows), lambda: (0, 0)),
                pl.BlockSpec((1, 1), lambda: (0, 0)),
                pl.BlockSpec((1, tail), lambda: (0, 0)),
                pl.BlockSpec((tail, t), lambda: (0, 0)),
            ],
            out_specs=(
                pl.BlockSpec((2, 16), lambda: (0, 0)),
                pl.BlockSpec((1, t), lambda: (0, 0)),
            ),
        )(w2d, k2d, wt, xtail)
        out = _make_xt_kernel(n, t, bc, nb)(stats, tailp, w, x.T)
    else:
        stats = pl.pallas_call(
            _stats_kernel,
            out_shape=jax.ShapeDtypeStruct((2, 16), jnp.float32),
            in_specs=[
                pl.BlockSpec((n // rows, rows), lambda: (0, 0)),
                pl.BlockSpec((1, 1), lambda: (0, 0)),
            ],
            out_specs=pl.BlockSpec((2, 16), lambda: (0, 0)),
        )(w2d, k2d)
        b = _pick_block(n)
        out = pl.pallas_call(
            _wsum_tc_grid_kernel,
            grid=(n // b,),
            out_shape=jax.ShapeDtypeStruct((1, t), jnp.float32),
            in_specs=[
                pl.BlockSpec((2, 16), lambda i: (0, 0)),
                pl.BlockSpec((1, 1, b), lambda i: (i, 0, 0)),
                pl.BlockSpec((b, t), lambda i: (i, 0)),
            ],
            out_specs=pl.BlockSpec((1, t), lambda i: (0, 0)),
        )(stats, w.reshape(n // b, 1, b), x)

    return out.reshape(t)
